# Initial kernel scaffold; baseline (speedup 1.0000x reference)
import jax
import jax.numpy as jnp
from jax.experimental import pallas as pl


def kernel(edge_index, edge_type, target_idx, W1, root1, b1, W2, root2, b2, Wlin, blin):
    raise NotImplementedError("write your pallas kernel here")


# speedup vs baseline: 1.0002x; 1.0002x over previous
import functools

import jax
import jax.numpy as jnp
from jax import lax
from jax.experimental import pallas as pl
from jax.experimental.pallas import tpu as pltpu
from jax.experimental.pallas import tpu_sc as plsc

_N = 10000
_E = 320000
_R = 8
_H = 128
_C = 8
_T = 4000

_NC = 2
_NS = 16
_NW = _NC * _NS
_TPAD = 4096


def _sc_gather_rows(table, idx_pad, n_rows_pad):
  per = n_rows_pad // _NW
  mesh = plsc.VectorSubcoreMesh(core_axis_name="c", subcore_axis_name="s")

  @functools.partial(
      pl.kernel,
      out_type=jax.ShapeDtypeStruct((n_rows_pad, _H), jnp.float32),
      mesh=mesh,
      scratch_types=[
          pltpu.VMEM((per,), jnp.int32),
          pltpu.VMEM((per, _H), jnp.float32),
          pltpu.SemaphoreType.DMA,
      ],
  )
  def k(table_hbm, idx_hbm, out_hbm, idx_v, rows_v, sem):
    wid = lax.axis_index("s") * _NC + lax.axis_index("c")
    base = wid * per
    pltpu.sync_copy(idx_hbm.at[pl.ds(base, per)], idx_v)
    pltpu.async_copy(table_hbm.at[idx_v], rows_v, sem).wait()
    pltpu.sync_copy(rows_v, out_hbm.at[pl.ds(base, per)])

  return k(table, idx_pad)


def kernel(edge_index, edge_type, target_idx, W1, root1, b1, W2, root2, b2,
           Wlin, blin):
  src = edge_index[0]
  dst = edge_index[1]

  deg = jnp.zeros((_R, _N), jnp.float32).at[edge_type, dst].add(1.0)
  norm = 1.0 / jnp.maximum(deg[edge_type, dst], 1.0)

  msg1 = W1[edge_type, src]
  agg1 = jnp.zeros((_N, _H), jnp.float32).at[dst].add(msg1 * norm[:, None])
  h = jax.nn.elu(agg1 + root1 + b1)

  xW = jnp.einsum('nh,rhk->rnk', h, W2)
  msg2 = xW[edge_type, src]
  agg2 = jnp.zeros((_N, _H), jnp.float32).at[dst].add(msg2 * norm[:, None])
  x = jax.nn.elu(agg2 + h @ root2 + b2)

  idx_pad = jnp.zeros((_TPAD,), jnp.int32).at[:_T].set(target_idx)
  xt = _sc_gather_rows(x, idx_pad, _TPAD)[:_T]

  return xt @ Wlin + blin


# speedup vs baseline: 16.6755x; 16.6728x over previous
import functools

import jax
import jax.numpy as jnp
from jax import lax
from jax.experimental import pallas as pl
from jax.experimental.pallas import tpu as pltpu
from jax.experimental.pallas import tpu_sc as plsc

_N = 10000
_E = 320000
_R = 8
_H = 128
_C = 8
_T = 4000

_NC = 2
_NS = 16
_NW = _NC * _NS
_TPAD = 4096
_RNP = 81920
_NP = 10240
_EPW = _E // _NW

_MESH = dict(core_axis_name="c", subcore_axis_name="s")
_F32 = jnp.float32
_I32 = jnp.int32


def _wid():
  return lax.axis_index("s") * _NC + lax.axis_index("c")


def _vbroadcast(v, lane):
  idx = jnp.full((16, 1), lane, _I32)
  return lax.gather(
      v, idx,
      dimension_numbers=lax.GatherDimensionNumbers(
          offset_dims=(), collapsed_slice_dims=(0,), start_index_map=(0,)),
      slice_sizes=(1,),
      mode=lax.GatherScatterMode.PROMISE_IN_BOUNDS)


def _sc_degree(et, dst):
  CH = 128
  SL = _RNP // _NS
  mesh = plsc.VectorSubcoreMesh(**_MESH)

  @functools.partial(
      pl.kernel,
      out_type=jax.ShapeDtypeStruct((_NC, _RNP), _F32),
      mesh=mesh,
      scratch_types=[
          pltpu.VMEM((CH,), _I32),
          pltpu.VMEM((CH,), _I32),
          pltpu.VMEM((CH,), _I32),
          pltpu.VMEM((16,), _I32),
          pltpu.VMEM((CH,), _F32),
          pltpu.VMEM((CH,), _F32),
          pltpu.VMEM_SHARED((_RNP,), _F32),
      ],
  )
  def k(et_hbm, dst_hbm, out_hbm, et_v, dst_v, fidx_v, fidx2_v, ones_v, zb_v,
        sh_hist):
    c = lax.axis_index("c")
    s = lax.axis_index("s")
    wid = _wid()
    for i in range(CH // 16):
      ones_v[pl.ds(i * 16, 16)] = jnp.ones((16,), _F32)
      zb_v[pl.ds(i * 16, 16)] = jnp.zeros((16,), _F32)

    def zc(i, carry):
      pltpu.sync_copy(zb_v, sh_hist.at[pl.ds(s * SL + i * CH, CH)])
      return carry
    lax.fori_loop(0, SL // CH, zc, 0)
    plsc.subcore_barrier()

    def chunk(g, carry):
      base = wid * _EPW + g * CH
      pltpu.sync_copy(et_hbm.at[pl.ds(base, CH)], et_v)
      pltpu.sync_copy(dst_hbm.at[pl.ds(base, CH)], dst_v)
      for i in range(CH // 16):
        e16 = et_v[pl.ds(i * 16, 16)]
        d16 = dst_v[pl.ds(i * 16, 16)]
        fidx_v[pl.ds(i * 16, 16)] = e16 * _N + d16
      pltpu.sync_copy(ones_v, sh_hist.at[fidx_v], add=True)
      return carry
    lax.fori_loop(0, _EPW // CH, chunk, 0)

    tbase = wid * _EPW + (_EPW // CH) * CH
    pltpu.sync_copy(et_hbm.at[pl.ds(tbase, 16)], et_v.at[pl.ds(0, 16)])
    pltpu.sync_copy(dst_hbm.at[pl.ds(tbase, 16)], dst_v.at[pl.ds(0, 16)])
    fidx2_v[...] = et_v[pl.ds(0, 16)] * _N + dst_v[pl.ds(0, 16)]
    pltpu.sync_copy(ones_v.at[pl.ds(0, 16)], sh_hist.at[fidx2_v], add=True)

    plsc.subcore_barrier()
    pltpu.sync_copy(sh_hist.at[pl.ds(s * SL, SL)],
                    out_hbm.at[c, pl.ds(s * SL, SL)])

  return k(et, dst)


def _sc_norm(deg_parts, et, src, dst):
  CH = 128
  SL = _RNP // _NS
  mesh = plsc.VectorSubcoreMesh(**_MESH)

  @functools.partial(
      pl.kernel,
      out_type=(jax.ShapeDtypeStruct((_E,), _F32),
                jax.ShapeDtypeStruct((_E,), _I32)),
      mesh=mesh,
      scratch_types=[
          pltpu.VMEM((SL,), _F32),
          pltpu.VMEM((SL,), _F32),
          pltpu.VMEM((CH,), _I32),
          pltpu.VMEM((CH,), _I32),
          pltpu.VMEM((CH,), _I32),
          pltpu.VMEM((CH,), _I32),
          pltpu.VMEM((CH,), _F32),
          pltpu.VMEM((CH,), _I32),
          pltpu.VMEM_SHARED((_RNP,), _F32),
      ],
  )
  def k(deg_hbm, et_hbm, src_hbm, dst_hbm, norm_out, flat_out,
        d0_v, d1_v, et_v, src_v, dst_v, gidx_v, nbuf_v, fbuf_v, sh_recip):
    s = lax.axis_index("s")
    wid = _wid()
    pltpu.sync_copy(deg_hbm.at[0, pl.ds(s * SL, SL)], d0_v)
    pltpu.sync_copy(deg_hbm.at[1, pl.ds(s * SL, SL)], d1_v)

    def rrow(i, carry):
      v = d0_v[pl.ds(i * 16, 16)] + d1_v[pl.ds(i * 16, 16)]
      d0_v[pl.ds(i * 16, 16)] = 1.0 / jnp.maximum(v, 1.0)
      return carry
    lax.fori_loop(0, SL // 16, rrow, 0)
    pltpu.sync_copy(d0_v, sh_recip.at[pl.ds(s * SL, SL)])
    plsc.subcore_barrier()

    def do_block(base, n):
      pltpu.sync_copy(et_hbm.at[pl.ds(base, n)], et_v.at[pl.ds(0, n)])
      pltpu.sync_copy(src_hbm.at[pl.ds(base, n)], src_v.at[pl.ds(0, n)])
      pltpu.sync_copy(dst_hbm.at[pl.ds(base, n)], dst_v.at[pl.ds(0, n)])
      for i in range(n // 16):
        e16 = et_v[pl.ds(i * 16, 16)]
        s16 = src_v[pl.ds(i * 16, 16)]
        d16 = dst_v[pl.ds(i * 16, 16)]
        gidx_v[pl.ds(i * 16, 16)] = e16 * _N + d16
        fbuf_v[pl.ds(i * 16, 16)] = e16 * _N + s16
      pltpu.sync_copy(sh_recip.at[gidx_v.at[pl.ds(0, n)]],
                      nbuf_v.at[pl.ds(0, n)])
      pltpu.sync_copy(nbuf_v.at[pl.ds(0, n)], norm_out.at[pl.ds(base, n)])
      pltpu.sync_copy(fbuf_v.at[pl.ds(0, n)], flat_out.at[pl.ds(base, n)])

    def chunk(g, carry):
      do_block(wid * _EPW + g * CH, CH)
      return carry
    lax.fori_loop(0, _EPW // CH, chunk, 0)
    do_block(wid * _EPW + (_EPW // CH) * CH, _EPW % CH)

  return k(deg_parts, et, src, dst)


def _sc_agg(table, flat1, dst, norm):
  CH = 128
  NCHUNK = 78
  RPT = _NP // _NS
  mesh = plsc.VectorSubcoreMesh(**_MESH)

  @functools.partial(
      pl.kernel,
      out_type=jax.ShapeDtypeStruct((_NC, _NP, _H), _F32),
      mesh=mesh,
      scratch_types=[
          pltpu.VMEM((CH,), _I32),
          pltpu.VMEM((CH,), _I32),
          pltpu.VMEM((CH,), _F32),
          pltpu.VMEM((CH, _H), _F32),
          pltpu.VMEM((16,), _I32),
          pltpu.VMEM((16,), _I32),
          pltpu.VMEM((16,), _F32),
          pltpu.VMEM((16, _H), _F32),
          pltpu.VMEM((64, _H), _F32),
          pltpu.SemaphoreType.DMA,
          pltpu.VMEM_SHARED((_NP, _H), _F32),
      ],
  )
  def k(table_hbm, flat_hbm, dst_hbm, norm_hbm, out_hbm,
        fidx_v, didx_v, norm_v, rows_v, fidx2_v, didx2_v, norm2_v, rows2_v,
        zb_v, sem, sh_acc):
    c = lax.axis_index("c")
    s = lax.axis_index("s")
    wid = _wid()
    zero16 = jnp.zeros((16,), _F32)
    for i in range(64):
      for j in range(8):
        zb_v[i, pl.ds(j * 16, 16)] = zero16

    def zc(j, carry):
      pltpu.sync_copy(zb_v, sh_acc.at[pl.ds(s * RPT + j * 64, 64)])
      return carry
    lax.fori_loop(0, RPT // 64, zc, 0)
    plsc.subcore_barrier()

    def scale_rows(rows_ref, nrm_ref, nrows):
      def sgrp(g16, carry):
        nv = nrm_ref[pl.ds(g16 * 16, 16)]
        for l in range(16):
          kr = g16 * 16 + l
          bc = _vbroadcast(nv, l)
          for j in range(8):
            rows_ref[kr, pl.ds(j * 16, 16)] = (
                rows_ref[kr, pl.ds(j * 16, 16)] * bc)
        return carry
      lax.fori_loop(0, nrows // 16, sgrp, 0)

    def chunk(g, carry):
      base = wid * _EPW + g * CH
      pltpu.sync_copy(flat_hbm.at[pl.ds(base, CH)], fidx_v)
      pltpu.sync_copy(dst_hbm.at[pl.ds(base, CH)], didx_v)
      pltpu.sync_copy(norm_hbm.at[pl.ds(base, CH)], norm_v)
      pltpu.async_copy(table_hbm.at[fidx_v], rows_v, sem).wait()
      scale_rows(rows_v, norm_v, CH)
      pltpu.sync_copy(rows_v, sh_acc.at[didx_v], add=True)
      return carry
    lax.fori_loop(0, NCHUNK, chunk, 0)

    tbase = wid * _EPW + NCHUNK * CH
    pltpu.sync_copy(flat_hbm.at[pl.ds(tbase, 16)], fidx2_v)
    pltpu.sync_copy(dst_hbm.at[pl.ds(tbase, 16)], didx2_v)
    pltpu.sync_copy(norm_hbm.at[pl.ds(tbase, 16)], norm2_v)
    pltpu.async_copy(table_hbm.at[fidx2_v], rows2_v, sem).wait()
    scale_rows(rows2_v, norm2_v, 16)
    pltpu.sync_copy(rows2_v, sh_acc.at[didx2_v], add=True)

    plsc.subcore_barrier()
    pltpu.sync_copy(sh_acc.at[pl.ds(s * RPT, RPT)],
                    out_hbm.at[c, pl.ds(s * RPT, RPT)])

  return k(table, flat1, dst, norm)


def _sc_gather_rows(table, idx_pad, n_rows_pad):
  per = n_rows_pad // _NW
  mesh = plsc.VectorSubcoreMesh(**_MESH)

  @functools.partial(
      pl.kernel,
      out_type=jax.ShapeDtypeStruct((n_rows_pad, _H), _F32),
      mesh=mesh,
      scratch_types=[
          pltpu.VMEM((per,), _I32),
          pltpu.VMEM((per, _H), _F32),
          pltpu.SemaphoreType.DMA,
      ],
  )
  def k(table_hbm, idx_hbm, out_hbm, idx_v, rows_v, sem):
    base = _wid() * per
    pltpu.sync_copy(idx_hbm.at[pl.ds(base, per)], idx_v)
    pltpu.async_copy(table_hbm.at[idx_v], rows_v, sem).wait()
    pltpu.sync_copy(rows_v, out_hbm.at[pl.ds(base, per)])

  return k(table, idx_pad)


def _elu(x):
  return jnp.where(x > 0, x, jnp.exp(x) - 1.0)


def _tc_conv(p1, root1, b1, W2, root2):
  BN = 400

  def body(p_ref, r1_ref, b1_ref, w2_ref, rt2_ref, xw_ref, h2_ref):
    h = p_ref[0] + p_ref[1] + r1_ref[...] + b1_ref[...]
    h = _elu(h)
    for r in range(_R):
      xw_ref[r] = jnp.dot(h, w2_ref[r], preferred_element_type=_F32)
    h2_ref[...] = jnp.dot(h, rt2_ref[...], preferred_element_type=_F32)

  return pl.pallas_call(
      body,
      grid=(_N // BN,),
      in_specs=[
          pl.BlockSpec((2, BN, _H), lambda i: (0, i, 0)),
          pl.BlockSpec((BN, _H), lambda i: (i, 0)),
          pl.BlockSpec((1, _H), lambda i: (0, 0)),
          pl.BlockSpec((_R, _H, _H), lambda i: (0, 0, 0)),
          pl.BlockSpec((_H, _H), lambda i: (0, 0)),
      ],
      out_specs=[
          pl.BlockSpec((_R, BN, _H), lambda i: (0, i, 0)),
          pl.BlockSpec((BN, _H), lambda i: (i, 0)),
      ],
      out_shape=[
          jax.ShapeDtypeStruct((_R, _N, _H), _F32),
          jax.ShapeDtypeStruct((_N, _H), _F32),
      ],
  )(p1, root1, b1, W2, root2)


def _tc_final(p2, h2, b2, wlin_pad, blin_pad):
  BN = 400

  def body(p_ref, h2_ref, b2_ref, wl_ref, bl_ref, y_ref):
    x = _elu(p_ref[0] + p_ref[1] + h2_ref[...] + b2_ref[...])
    y_ref[...] = (jnp.dot(x, wl_ref[...], preferred_element_type=_F32)
                  + bl_ref[...])

  return pl.pallas_call(
      body,
      grid=(_N // BN,),
      in_specs=[
          pl.BlockSpec((2, BN, _H), lambda i: (0, i, 0)),
          pl.BlockSpec((BN, _H), lambda i: (i, 0)),
          pl.BlockSpec((1, _H), lambda i: (0, 0)),
          pl.BlockSpec((_H, _H), lambda i: (0, 0)),
          pl.BlockSpec((1, _H), lambda i: (0, 0)),
      ],
      out_specs=pl.BlockSpec((BN, _H), lambda i: (i, 0)),
      out_shape=jax.ShapeDtypeStruct((_N, _H), _F32),
  )(p2, h2, b2, wlin_pad, blin_pad)


def kernel(edge_index, edge_type, target_idx, W1, root1, b1, W2, root2, b2,
           Wlin, blin):
  src = edge_index[0]
  dst = edge_index[1]

  deg_parts = _sc_degree(edge_type, dst)
  norm, flat1 = _sc_norm(deg_parts, edge_type, src, dst)

  p1 = _sc_agg(W1.reshape(_R * _N, _H), flat1, dst, norm)
  xW, h2 = _tc_conv(p1, root1, b1.reshape(1, _H), W2, root2)
  p2 = _sc_agg(xW.reshape(_R * _N, _H), flat1, dst, norm)

  wlin_pad = jnp.zeros((_H, _H), _F32).at[:, :_C].set(Wlin)
  blin_pad = jnp.zeros((1, _H), _F32).at[0, :_C].set(blin)
  y = _tc_final(p2, h2, b2.reshape(1, _H), wlin_pad, blin_pad)

  idx_pad = jnp.zeros((_TPAD,), _I32).at[:_T].set(target_idx)
  xt = _sc_gather_rows(y, idx_pad, _TPAD)

  return xt[:_T, :_C]


# speedup vs baseline: 23.8726x; 1.4316x over previous
import functools

import jax
import jax.numpy as jnp
from jax import lax
from jax.experimental import pallas as pl
from jax.experimental.pallas import tpu as pltpu
from jax.experimental.pallas import tpu_sc as plsc

_N = 10000
_E = 320000
_R = 8
_H = 128
_C = 8
_T = 4000

_NC = 2
_NS = 16
_NW = _NC * _NS
_TPAD = 4096
_RNP = 81920
_NP = 10240
_EPW = _E // _NW

_MESH = dict(core_axis_name="c", subcore_axis_name="s")
_F32 = jnp.float32
_I32 = jnp.int32


def _wid():
  return lax.axis_index("s") * _NC + lax.axis_index("c")


def _vbroadcast(v, lane):
  idx = jnp.full((16, 1), lane, _I32)
  return lax.gather(
      v, idx,
      dimension_numbers=lax.GatherDimensionNumbers(
          offset_dims=(), collapsed_slice_dims=(0,), start_index_map=(0,)),
      slice_sizes=(1,),
      mode=lax.GatherScatterMode.PROMISE_IN_BOUNDS)


def _sc_degree(et, dst):
  CH = 128
  SL = _RNP // _NS
  mesh = plsc.VectorSubcoreMesh(**_MESH)

  @functools.partial(
      pl.kernel,
      out_type=jax.ShapeDtypeStruct((_NC, _RNP), _F32),
      mesh=mesh,
      scratch_types=[
          pltpu.VMEM((CH,), _I32),
          pltpu.VMEM((CH,), _I32),
          pltpu.VMEM((CH,), _I32),
          pltpu.VMEM((16,), _I32),
          pltpu.VMEM((CH,), _F32),
          pltpu.VMEM((CH,), _F32),
          pltpu.VMEM_SHARED((_RNP,), _F32),
      ],
  )
  def k(et_hbm, dst_hbm, out_hbm, et_v, dst_v, fidx_v, fidx2_v, ones_v, zb_v,
        sh_hist):
    c = lax.axis_index("c")
    s = lax.axis_index("s")
    wid = _wid()
    for i in range(CH // 16):
      ones_v[pl.ds(i * 16, 16)] = jnp.ones((16,), _F32)
      zb_v[pl.ds(i * 16, 16)] = jnp.zeros((16,), _F32)

    def zc(i, carry):
      pltpu.sync_copy(zb_v, sh_hist.at[pl.ds(s * SL + i * CH, CH)])
      return carry
    lax.fori_loop(0, SL // CH, zc, 0)
    plsc.subcore_barrier()

    def chunk(g, carry):
      base = wid * _EPW + g * CH
      pltpu.sync_copy(et_hbm.at[pl.ds(base, CH)], et_v)
      pltpu.sync_copy(dst_hbm.at[pl.ds(base, CH)], dst_v)
      for i in range(CH // 16):
        e16 = et_v[pl.ds(i * 16, 16)]
        d16 = dst_v[pl.ds(i * 16, 16)]
        fidx_v[pl.ds(i * 16, 16)] = e16 * _N + d16
      pltpu.sync_copy(ones_v, sh_hist.at[fidx_v], add=True)
      return carry
    lax.fori_loop(0, _EPW // CH, chunk, 0)

    tbase = wid * _EPW + (_EPW // CH) * CH
    pltpu.sync_copy(et_hbm.at[pl.ds(tbase, 16)], et_v.at[pl.ds(0, 16)])
    pltpu.sync_copy(dst_hbm.at[pl.ds(tbase, 16)], dst_v.at[pl.ds(0, 16)])
    fidx2_v[...] = et_v[pl.ds(0, 16)] * _N + dst_v[pl.ds(0, 16)]
    pltpu.sync_copy(ones_v.at[pl.ds(0, 16)], sh_hist.at[fidx2_v], add=True)

    plsc.subcore_barrier()
    pltpu.sync_copy(sh_hist.at[pl.ds(s * SL, SL)],
                    out_hbm.at[c, pl.ds(s * SL, SL)])

  return k(et, dst)


def _sc_norm(deg_parts, et, src, dst):
  CH = 128
  SL = _RNP // _NS
  mesh = plsc.VectorSubcoreMesh(**_MESH)

  @functools.partial(
      pl.kernel,
      out_type=(jax.ShapeDtypeStruct((_E,), _F32),
                jax.ShapeDtypeStruct((_E,), _I32)),
      mesh=mesh,
      scratch_types=[
          pltpu.VMEM((SL,), _F32),
          pltpu.VMEM((SL,), _F32),
          pltpu.VMEM((CH,), _I32),
          pltpu.VMEM((CH,), _I32),
          pltpu.VMEM((CH,), _I32),
          pltpu.VMEM((CH,), _I32),
          pltpu.VMEM((CH,), _F32),
          pltpu.VMEM((CH,), _I32),
          pltpu.VMEM_SHARED((_RNP,), _F32),
      ],
  )
  def k(deg_hbm, et_hbm, src_hbm, dst_hbm, norm_out, flat_out,
        d0_v, d1_v, et_v, src_v, dst_v, gidx_v, nbuf_v, fbuf_v, sh_recip):
    s = lax.axis_index("s")
    wid = _wid()
    pltpu.sync_copy(deg_hbm.at[0, pl.ds(s * SL, SL)], d0_v)
    pltpu.sync_copy(deg_hbm.at[1, pl.ds(s * SL, SL)], d1_v)

    def rrow(i, carry):
      v = d0_v[pl.ds(i * 16, 16)] + d1_v[pl.ds(i * 16, 16)]
      d0_v[pl.ds(i * 16, 16)] = 1.0 / jnp.maximum(v, 1.0)
      return carry
    lax.fori_loop(0, SL // 16, rrow, 0)
    pltpu.sync_copy(d0_v, sh_recip.at[pl.ds(s * SL, SL)])
    plsc.subcore_barrier()

    def do_block(base, n):
      pltpu.sync_copy(et_hbm.at[pl.ds(base, n)], et_v.at[pl.ds(0, n)])
      pltpu.sync_copy(src_hbm.at[pl.ds(base, n)], src_v.at[pl.ds(0, n)])
      pltpu.sync_copy(dst_hbm.at[pl.ds(base, n)], dst_v.at[pl.ds(0, n)])
      for i in range(n // 16):
        e16 = et_v[pl.ds(i * 16, 16)]
        s16 = src_v[pl.ds(i * 16, 16)]
        d16 = dst_v[pl.ds(i * 16, 16)]
        gidx_v[pl.ds(i * 16, 16)] = e16 * _N + d16
        fbuf_v[pl.ds(i * 16, 16)] = e16 * _N + s16
      pltpu.sync_copy(sh_recip.at[gidx_v.at[pl.ds(0, n)]],
                      nbuf_v.at[pl.ds(0, n)])
      pltpu.sync_copy(nbuf_v.at[pl.ds(0, n)], norm_out.at[pl.ds(base, n)])
      pltpu.sync_copy(fbuf_v.at[pl.ds(0, n)], flat_out.at[pl.ds(base, n)])

    def chunk(g, carry):
      do_block(wid * _EPW + g * CH, CH)
      return carry
    lax.fori_loop(0, _EPW // CH, chunk, 0)
    do_block(wid * _EPW + (_EPW // CH) * CH, _EPW % CH)

  return k(deg_parts, et, src, dst)


def _sc_agg(table, flat1, dst, norm):
  CH = 128
  NCK = 78
  RPT = _NP // _NS
  mesh = plsc.VectorSubcoreMesh(**_MESH)

  @functools.partial(
      pl.kernel,
      out_type=jax.ShapeDtypeStruct((_NC, _NP, _H), _F32),
      mesh=mesh,
      scratch_types=[
          pltpu.VMEM((4, CH), _I32),
          pltpu.VMEM((4, CH), _I32),
          pltpu.VMEM((4, CH), _F32),
          pltpu.VMEM((2, CH, _H), _F32),
          pltpu.VMEM((64, _H), _F32),
          pltpu.SemaphoreType.DMA,
          pltpu.SemaphoreType.DMA,
          pltpu.SemaphoreType.DMA,
          pltpu.SemaphoreType.DMA,
          pltpu.SemaphoreType.DMA,
          pltpu.VMEM_SHARED((_NP, _H), _F32),
      ],
  )
  def k(table_hbm, flat_hbm, dst_hbm, norm_hbm, out_hbm,
        fidx_v, didx_v, norm_v, rows_v, zb_v,
        gsem0, gsem1, ssem0, ssem1, isem, sh_acc):
    c = lax.axis_index("c")
    s = lax.axis_index("s")
    wid = _wid()
    gsem = (gsem0, gsem1)
    ssem = (ssem0, ssem1)
    zero16 = jnp.zeros((16,), _F32)
    for i in range(64):
      for j in range(8):
        zb_v[i, pl.ds(j * 16, 16)] = zero16

    def zc(j, carry):
      pltpu.sync_copy(zb_v, sh_acc.at[pl.ds(s * RPT + j * 64, 64)])
      return carry
    lax.fori_loop(0, RPT // 64, zc, 0)
    plsc.subcore_barrier()

    def scale_rows(b, q):
      def sgrp(g16, carry):
        nv = norm_v[q, pl.ds(g16 * 16, 16)]
        for l in range(16):
          kr = g16 * 16 + l
          bc = _vbroadcast(nv, l)
          for j in range(8):
            rows_v[b, kr, pl.ds(j * 16, 16)] = (
                rows_v[b, kr, pl.ds(j * 16, 16)] * bc)
        return carry
      lax.fori_loop(0, CH // 16, sgrp, 0)

    def idx_load(g, q, sync):
      base = (wid * NCK + g) * CH
      if sync:
        pltpu.sync_copy(flat_hbm.at[pl.ds(base, CH)], fidx_v.at[q])
        pltpu.sync_copy(dst_hbm.at[pl.ds(base, CH)], didx_v.at[q])
        pltpu.sync_copy(norm_hbm.at[pl.ds(base, CH)], norm_v.at[q])
      else:
        pltpu.async_copy(flat_hbm.at[pl.ds(base, CH)], fidx_v.at[q], isem)
        pltpu.async_copy(dst_hbm.at[pl.ds(base, CH)], didx_v.at[q], isem)
        pltpu.async_copy(norm_hbm.at[pl.ds(base, CH)], norm_v.at[q], isem)

    def idx_wait(q):
      pltpu.make_async_copy(flat_hbm.at[pl.ds(0, CH)], fidx_v.at[q],
                            isem).wait()
      pltpu.make_async_copy(dst_hbm.at[pl.ds(0, CH)], didx_v.at[q],
                            isem).wait()
      pltpu.make_async_copy(norm_hbm.at[pl.ds(0, CH)], norm_v.at[q],
                            isem).wait()

    def gather_start(g, b, q):
      pltpu.async_copy(table_hbm.at[fidx_v.at[q]], rows_v.at[b], gsem[b])

    idx_load(0, 0, True)
    idx_load(1, 1, False)
    gather_start(0, 0, 0)

    def body2(gp, carry):
      for half in range(2):
        b = half
        g = gp * 2 + half
        q = g % 4
        pltpu.make_async_copy(table_hbm.at[fidx_v.at[q]], rows_v.at[b],
                              gsem[b]).wait()
        scale_rows(b, q)

        @pl.when(g >= 1)
        def _():
          pltpu.make_async_copy(rows_v.at[1 - b], sh_acc.at[didx_v.at[0]],
                                ssem[1 - b]).wait()

        pltpu.async_copy(rows_v.at[b], sh_acc.at[didx_v.at[q]], ssem[b],
                         add=True)

        @pl.when(g <= NCK - 3)
        def _():
          idx_load(g + 2, (g + 2) % 4, False)

        @pl.when(g <= NCK - 2)
        def _():
          idx_wait((g + 1) % 4)
          gather_start(g + 1, 1 - b, (g + 1) % 4)
      return carry
    lax.fori_loop(0, NCK // 2, body2, 0)
    pltpu.make_async_copy(rows_v.at[1], sh_acc.at[didx_v.at[0]],
                          ssem[1]).wait()

    @pl.when(wid < 4)
    def _():
      base = (NCK * _NW + wid) * CH
      pltpu.sync_copy(flat_hbm.at[pl.ds(base, CH)], fidx_v.at[0])
      pltpu.sync_copy(dst_hbm.at[pl.ds(base, CH)], didx_v.at[0])
      pltpu.sync_copy(norm_hbm.at[pl.ds(base, CH)], norm_v.at[0])
      pltpu.async_copy(table_hbm.at[fidx_v.at[0]], rows_v.at[0],
                       gsem0).wait()
      scale_rows(0, 0)
      pltpu.async_copy(rows_v.at[0], sh_acc.at[didx_v.at[0]], ssem0,
                       add=True).wait()

    plsc.subcore_barrier()
    pltpu.sync_copy(sh_acc.at[pl.ds(s * RPT, RPT)],
                    out_hbm.at[c, pl.ds(s * RPT, RPT)])

  return k(table, flat1, dst, norm)


def _sc_gather_rows(table, idx_pad, n_rows_pad):
  per = n_rows_pad // _NW
  mesh = plsc.VectorSubcoreMesh(**_MESH)

  @functools.partial(
      pl.kernel,
      out_type=jax.ShapeDtypeStruct((n_rows_pad, _H), _F32),
      mesh=mesh,
      scratch_types=[
          pltpu.VMEM((per,), _I32),
          pltpu.VMEM((per, _H), _F32),
          pltpu.SemaphoreType.DMA,
      ],
  )
  def k(table_hbm, idx_hbm, out_hbm, idx_v, rows_v, sem):
    base = _wid() * per
    pltpu.sync_copy(idx_hbm.at[pl.ds(base, per)], idx_v)
    pltpu.async_copy(table_hbm.at[idx_v], rows_v, sem).wait()
    pltpu.sync_copy(rows_v, out_hbm.at[pl.ds(base, per)])

  return k(table, idx_pad)


def _elu(x):
  return jnp.where(x > 0, x, jnp.exp(x) - 1.0)


def _tc_conv(p1, root1, b1, W2, root2):
  BN = 400

  def body(p_ref, r1_ref, b1_ref, w2_ref, rt2_ref, xw_ref, h2_ref):
    h = p_ref[0] + p_ref[1] + r1_ref[...] + b1_ref[...]
    h = _elu(h)
    for r in range(_R):
      xw_ref[r] = jnp.dot(h, w2_ref[r], preferred_element_type=_F32)
    h2_ref[...] = jnp.dot(h, rt2_ref[...], preferred_element_type=_F32)

  return pl.pallas_call(
      body,
      grid=(_N // BN,),
      in_specs=[
          pl.BlockSpec((2, BN, _H), lambda i: (0, i, 0)),
          pl.BlockSpec((BN, _H), lambda i: (i, 0)),
          pl.BlockSpec((1, _H), lambda i: (0, 0)),
          pl.BlockSpec((_R, _H, _H), lambda i: (0, 0, 0)),
          pl.BlockSpec((_H, _H), lambda i: (0, 0)),
      ],
      out_specs=[
          pl.BlockSpec((_R, BN, _H), lambda i: (0, i, 0)),
          pl.BlockSpec((BN, _H), lambda i: (i, 0)),
      ],
      out_shape=[
          jax.ShapeDtypeStruct((_R, _N, _H), _F32),
          jax.ShapeDtypeStruct((_N, _H), _F32),
      ],
  )(p1, root1, b1, W2, root2)


def _tc_final(p2, h2, b2, wlin_pad, blin_pad):
  BN = 400

  def body(p_ref, h2_ref, b2_ref, wl_ref, bl_ref, y_ref):
    x = _elu(p_ref[0] + p_ref[1] + h2_ref[...] + b2_ref[...])
    y_ref[...] = (jnp.dot(x, wl_ref[...], preferred_element_type=_F32)
                  + bl_ref[...])

  return pl.pallas_call(
      body,
      grid=(_N // BN,),
      in_specs=[
          pl.BlockSpec((2, BN, _H), lambda i: (0, i, 0)),
          pl.BlockSpec((BN, _H), lambda i: (i, 0)),
          pl.BlockSpec((1, _H), lambda i: (0, 0)),
          pl.BlockSpec((_H, _H), lambda i: (0, 0)),
          pl.BlockSpec((1, _H), lambda i: (0, 0)),
      ],
      out_specs=pl.BlockSpec((BN, _H), lambda i: (i, 0)),
      out_shape=jax.ShapeDtypeStruct((_N, _H), _F32),
  )(p2, h2, b2, wlin_pad, blin_pad)


def kernel(edge_index, edge_type, target_idx, W1, root1, b1, W2, root2, b2,
           Wlin, blin):
  src = edge_index[0]
  dst = edge_index[1]

  deg_parts = _sc_degree(edge_type, dst)
  norm, flat1 = _sc_norm(deg_parts, edge_type, src, dst)

  p1 = _sc_agg(W1.reshape(_R * _N, _H), flat1, dst, norm)
  xW, h2 = _tc_conv(p1, root1, b1.reshape(1, _H), W2, root2)
  p2 = _sc_agg(xW.reshape(_R * _N, _H), flat1, dst, norm)

  wlin_pad = jnp.zeros((_H, _H), _F32).at[:, :_C].set(Wlin)
  blin_pad = jnp.zeros((1, _H), _F32).at[0, :_C].set(blin)
  y = _tc_final(p2, h2, b2.reshape(1, _H), wlin_pad, blin_pad)

  idx_pad = jnp.zeros((_TPAD,), _I32).at[:_T].set(target_idx)
  xt = _sc_gather_rows(y, idx_pad, _TPAD)

  return xt[:_T, :_C]
